# Initial kernel scaffold; baseline (speedup 1.0000x reference)
import jax
import jax.numpy as jnp
from jax.experimental import pallas as pl


def kernel(x, emb):
    raise NotImplementedError("write your pallas kernel here")


# speedup vs baseline: 4.4512x; 4.4512x over previous
import jax
import jax.numpy as jnp
from jax.experimental import pallas as pl

_BLK_S = 256


def _pe_add_kernel(x_ref, emb_ref, o_ref):
    o_ref[...] = x_ref[...] + emb_ref[...][:, None, :]


def kernel(x, emb):
    S, B, D = x.shape
    grid = (S // _BLK_S,)
    return pl.pallas_call(
        _pe_add_kernel,
        grid=grid,
        in_specs=[
            pl.BlockSpec((_BLK_S, B, D), lambda i: (i, 0, 0)),
            pl.BlockSpec((_BLK_S, D), lambda i: (i, 0)),
        ],
        out_specs=pl.BlockSpec((_BLK_S, B, D), lambda i: (i, 0, 0)),
        out_shape=jax.ShapeDtypeStruct((S, B, D), x.dtype),
    )(x, emb)
